# baseline (speedup 1.0000x reference)
import functools

import jax
import jax.numpy as jnp
import numpy as np
from jax.experimental import pallas as pl

_BINS = [0.0, float(np.log1p(5.0)), float(np.log1p(25.0)),
         float(np.log1p(50.0)), float(np.log1p(100.0))]

_C = 512
_RB = 1024


def _body(yp_ref, yt_ref, out_ref):
    i = pl.program_id(0)

    @pl.when(i == 0)
    def _init():
        out_ref[...] = jnp.zeros_like(out_ref)

    t = yt_ref[...]
    d = yp_ref[...] - t
    d2 = d * d
    ge = [(t >= e).astype(jnp.float32) for e in _BINS]
    rows = []
    for b in range(4):
        ind = ge[b] - ge[b + 1]
        rows.append(jnp.sum(ind * d2, axis=0))
    for b in range(4):
        ind = ge[b] - ge[b + 1]
        rows.append(jnp.sum(ind, axis=0))
    out_ref[...] += jnp.stack(rows, axis=0)


@functools.partial(jax.jit, static_argnames=("interpret",))
def kernel(y_pred, y_true, interpret=False):
    n = y_pred.size
    rows = n // _C
    yp = y_pred.reshape(rows, _C)
    yt = y_true.reshape(rows, _C)
    acc = pl.pallas_call(
        _body,
        grid=(rows // _RB,),
        in_specs=[pl.BlockSpec((_RB, _C), lambda i: (i, 0))] * 2,
        out_specs=pl.BlockSpec((8, _C), lambda i: (0, 0)),
        out_shape=jax.ShapeDtypeStruct((8, _C), jnp.float32),
        interpret=interpret,
    )(yp, yt)
    q = jnp.sum(acc[0:4], axis=1)
    s = jnp.sum(acc[4:8], axis=1)
    mse = q / jnp.maximum(s, 1.0)
    valid = s > 0
    mses = jnp.where(valid, mse, 0.0)
    w = jnp.where(valid, 1.0 / jnp.maximum(s, 1.0), 0.0)
    w = w / jnp.sum(w)
    return jnp.sqrt(jnp.sum(w * mses) + 1e-8)
